# baseline (speedup 1.0000x reference)
import jax
import jax.numpy as jnp
from jax.experimental import pallas as pl
from jax.experimental.pallas import tpu as pltpu

_OUT_FEATURES = 10
_SPLIT = 4
_BATCH_TILE = 8192


def _linear_t_kernel(x0_ref, x1_ref, x2_ref, x3_ref, w_ref, b_ref, o_ref):
    tb = x0_ref.shape[0]
    for k, x_ref in enumerate((x0_ref, x1_ref, x2_ref, x3_ref)):
        acc = jnp.dot(x_ref[...], w_ref[...],
                      preferred_element_type=jnp.float32)
        acc = acc + b_ref[...]
        o_ref[:, k * tb:(k + 1) * tb] = jnp.transpose(acc).astype(o_ref.dtype)


def _linear_t_kernel_1(x_ref, w_ref, b_ref, o_ref):
    acc = jnp.dot(x_ref[...], w_ref[...], preferred_element_type=jnp.float32)
    acc = acc + b_ref[...]
    o_ref[...] = jnp.transpose(acc).astype(o_ref.dtype)


def kernel(x, w_padded, b_padded):
    B, in_f = x.shape
    out_f = _OUT_FEATURES
    w = w_padded[:, :out_f]
    b = b_padded[:, :out_f]

    tb = _BATCH_TILE
    group = _SPLIT * tb
    if B % group == 0:
        n_steps = B // group
        yt = pl.pallas_call(
            _linear_t_kernel,
            out_shape=jax.ShapeDtypeStruct((out_f, B), x.dtype),
            grid=(n_steps,),
            in_specs=[
                pl.BlockSpec((tb, in_f), lambda i: (_SPLIT * i + 0, 0)),
                pl.BlockSpec((tb, in_f), lambda i: (_SPLIT * i + 1, 0)),
                pl.BlockSpec((tb, in_f), lambda i: (_SPLIT * i + 2, 0)),
                pl.BlockSpec((tb, in_f), lambda i: (_SPLIT * i + 3, 0)),
                pl.BlockSpec((in_f, out_f), lambda i: (0, 0)),
                pl.BlockSpec((1, out_f), lambda i: (0, 0)),
            ],
            out_specs=pl.BlockSpec((out_f, group), lambda i: (0, i)),
            compiler_params=pltpu.CompilerParams(
                dimension_semantics=("parallel",)),
        )(x, x, x, x, w, b)
        return yt.T

    tb1 = min(4 * tb, B)
    b_rows = pl.cdiv(B, tb1) * tb1
    x_p = x if b_rows == B else jnp.pad(x, ((0, b_rows - B), (0, 0)))
    yt = pl.pallas_call(
        _linear_t_kernel_1,
        out_shape=jax.ShapeDtypeStruct((out_f, b_rows), x.dtype),
        grid=(b_rows // tb1,),
        in_specs=[
            pl.BlockSpec((tb1, in_f), lambda i: (i, 0)),
            pl.BlockSpec((in_f, out_f), lambda i: (0, 0)),
            pl.BlockSpec((1, out_f), lambda i: (0, 0)),
        ],
        out_specs=pl.BlockSpec((out_f, tb1), lambda i: (0, i)),
        compiler_params=pltpu.CompilerParams(
            dimension_semantics=("parallel",)),
    )(x_p, w, b)
    y = yt.T
    return y if b_rows == B else y[:B]


# speedup vs baseline: 1.0020x; 1.0020x over previous
import jax
import jax.numpy as jnp
from jax.experimental import pallas as pl
from jax.experimental.pallas import tpu as pltpu

_OUT_FEATURES = 10
_BATCH_TILE = 32768


def _linear_t_kernel(x_ref, w_ref, b_ref, o_ref):
    acc = jnp.dot(x_ref[...], w_ref[...], preferred_element_type=jnp.float32)
    acc = acc + b_ref[...]
    o_ref[...] = jnp.transpose(acc).astype(o_ref.dtype)


def kernel(x, w_padded, b_padded):
    B, in_f = x.shape
    out_f = _OUT_FEATURES
    w = w_padded[:, :out_f]
    b = b_padded[:, :out_f]

    tb = min(_BATCH_TILE, B)
    b_rows = pl.cdiv(B, tb) * tb
    x_p = x if b_rows == B else jnp.pad(x, ((0, b_rows - B), (0, 0)))

    yt = pl.pallas_call(
        _linear_t_kernel,
        out_shape=jax.ShapeDtypeStruct((out_f, b_rows), x.dtype),
        grid=(b_rows // tb,),
        in_specs=[
            pl.BlockSpec((tb, in_f), lambda i: (i, 0)),
            pl.BlockSpec((in_f, out_f), lambda i: (0, 0)),
            pl.BlockSpec((1, out_f), lambda i: (0, 0)),
        ],
        out_specs=pl.BlockSpec((out_f, tb), lambda i: (0, i)),
        compiler_params=pltpu.CompilerParams(
            dimension_semantics=("parallel",)),
    )(x_p, w, b)
    y = yt.T
    return y if b_rows == B else y[:B]
